# Initial kernel scaffold; baseline (speedup 1.0000x reference)
import jax
import jax.numpy as jnp
from jax.experimental import pallas as pl


def kernel(x, node_type, edge_index, edge_attr, params):
    raise NotImplementedError("write your pallas kernel here")


# speedup vs baseline: 2.7391x; 2.7391x over previous
import functools

import jax
import jax.numpy as jnp
from jax import lax
from jax.experimental import pallas as pl
from jax.experimental.pallas import tpu as pltpu
from jax.experimental.pallas import tpu_sc as plsc

H = 128
NC = 2
NS = 16
NW = NC * NS
EPS = 1e-5


def _ln(t, g, b):
    m = jnp.mean(t, axis=-1, keepdims=True)
    v = jnp.mean((t - m) * (t - m), axis=-1, keepdims=True)
    return (t - m) * lax.rsqrt(v + EPS) * g + b


def _full(block):
    return pl.BlockSpec(block, lambda i: tuple(0 for _ in block))


def _enc_body(x_ref, w1_ref, b1_ref, w2_ref, b2_ref, g_ref, bb_ref, o_ref):
    r = jnp.maximum(
        jnp.dot(x_ref[...], w1_ref[...], preferred_element_type=jnp.float32)
        + b1_ref[...], 0.0)
    t = jnp.dot(r, w2_ref[...], preferred_element_type=jnp.float32) + b2_ref[...]
    o_ref[...] = _ln(t, g_ref[...], bb_ref[...])


def _encoder(xin, W1, b1, W2, b2, g, bb, R):
    M, K = xin.shape
    return pl.pallas_call(
        _enc_body,
        grid=(M // R,),
        in_specs=[
            pl.BlockSpec((R, K), lambda i: (i, 0)),
            _full((K, H)), _full((1, H)), _full((H, H)), _full((1, H)),
            _full((1, H)), _full((1, H)),
        ],
        out_specs=pl.BlockSpec((R, H), lambda i: (i, 0)),
        out_shape=jax.ShapeDtypeStruct((M, H), jnp.float32),
    )(xin, W1, b1.reshape(1, H), W2, b2.reshape(1, H),
      g.reshape(1, H), bb.reshape(1, H))


def _proj_body(h_ref, ws_ref, wd_ref, p_ref, q_ref):
    h = h_ref[...]
    p_ref[...] = jnp.dot(h, ws_ref[...], preferred_element_type=jnp.float32)
    q_ref[...] = jnp.dot(h, wd_ref[...], preferred_element_type=jnp.float32)


def _proj(h, Ws, Wd, R):
    N = h.shape[0]
    return pl.pallas_call(
        _proj_body,
        grid=(N // R,),
        in_specs=[
            pl.BlockSpec((R, H), lambda i: (i, 0)),
            _full((H, H)), _full((H, H)),
        ],
        out_specs=[
            pl.BlockSpec((R, H), lambda i: (i, 0)),
            pl.BlockSpec((R, H), lambda i: (i, 0)),
        ],
        out_shape=[
            jax.ShapeDtypeStruct((N, H), jnp.float32),
            jax.ShapeDtypeStruct((N, H), jnp.float32),
        ],
    )(h, Ws, Wd)


def _edge_body(e_ref, gs_ref, w1_ref, b1_ref, w2_ref, b2_ref, g_ref, bb_ref,
               o_ref):
    e = e_ref[...]
    r = jnp.maximum(
        jnp.dot(e, w1_ref[...], preferred_element_type=jnp.float32)
        + gs_ref[...] + b1_ref[...], 0.0)
    t = jnp.dot(r, w2_ref[...], preferred_element_type=jnp.float32) + b2_ref[...]
    o_ref[...] = e + _ln(t, g_ref[...], bb_ref[...])


def _edge_update(e, gsum, W1e, b1, W2, b2, g, bb, R):
    E = e.shape[0]
    return pl.pallas_call(
        _edge_body,
        grid=(E // R,),
        in_specs=[
            pl.BlockSpec((R, H), lambda i: (i, 0)),
            pl.BlockSpec((R, H), lambda i: (i, 0)),
            _full((H, H)), _full((1, H)), _full((H, H)), _full((1, H)),
            _full((1, H)), _full((1, H)),
        ],
        out_specs=pl.BlockSpec((R, H), lambda i: (i, 0)),
        out_shape=jax.ShapeDtypeStruct((E, H), jnp.float32),
    )(e, gsum, W1e, b1.reshape(1, H), W2, b2.reshape(1, H),
      g.reshape(1, H), bb.reshape(1, H))


def _node_body(h_ref, a0_ref, a1_ref, wh_ref, wa_ref, b1_ref, w2_ref, b2_ref,
               g_ref, bb_ref, o_ref):
    h = h_ref[...]
    agg = a0_ref[...] + a1_ref[...]
    pre = (jnp.dot(h, wh_ref[...], preferred_element_type=jnp.float32)
           + jnp.dot(agg, wa_ref[...], preferred_element_type=jnp.float32)
           + b1_ref[...])
    r = jnp.maximum(pre, 0.0)
    t = jnp.dot(r, w2_ref[...], preferred_element_type=jnp.float32) + b2_ref[...]
    o_ref[...] = h + _ln(t, g_ref[...], bb_ref[...])


def _node_update(h, agg2, Wh, Wa, b1, W2, b2, g, bb, R):
    N = h.shape[0]
    nt = N // R
    return pl.pallas_call(
        _node_body,
        grid=(nt,),
        in_specs=[
            pl.BlockSpec((R, H), lambda i: (i, 0)),
            pl.BlockSpec((R, H), lambda i: (i, 0)),
            pl.BlockSpec((R, H), lambda i: (i + nt, 0)),
            _full((H, H)), _full((H, H)), _full((1, H)), _full((H, H)),
            _full((1, H)), _full((1, H)), _full((1, H)),
        ],
        out_specs=pl.BlockSpec((R, H), lambda i: (i, 0)),
        out_shape=jax.ShapeDtypeStruct((N, H), jnp.float32),
    )(h, agg2, agg2, Wh, Wa, b1.reshape(1, H), W2, b2.reshape(1, H),
      g.reshape(1, H), bb.reshape(1, H))


def _dec_body(h_ref, w1_ref, b1_ref, w2_ref, b2_ref, xr_ref, o_ref):
    r = jnp.maximum(
        jnp.dot(h_ref[...], w1_ref[...], preferred_element_type=jnp.float32)
        + b1_ref[...], 0.0)
    t = jnp.dot(r, w2_ref[...], preferred_element_type=jnp.float32) + b2_ref[...]
    o_ref[...] = t + xr_ref[...]


def _decoder(h, W1, b1, W2p, b2p, xres, R):
    N = h.shape[0]
    return pl.pallas_call(
        _dec_body,
        grid=(N // R,),
        in_specs=[
            pl.BlockSpec((R, H), lambda i: (i, 0)),
            _full((H, H)), _full((1, H)), _full((H, H)), _full((1, H)),
            pl.BlockSpec((R, H), lambda i: (i, 0)),
        ],
        out_specs=pl.BlockSpec((R, H), lambda i: (i, 0)),
        out_shape=jax.ShapeDtypeStruct((N, H), jnp.float32),
    )(h, W1, b1.reshape(1, H), W2p, b2p.reshape(1, H), xres)


_C = 128


def _sc_gather(p, q, src, dst):
    E = src.shape[0]
    per = E // NW
    nch = per // _C
    tail = per - nch * _C
    mesh = plsc.VectorSubcoreMesh(core_axis_name="c", subcore_axis_name="s")

    @functools.partial(
        pl.kernel,
        out_type=jax.ShapeDtypeStruct((E, H), jnp.float32),
        mesh=mesh,
        scratch_types=[
            pltpu.VMEM((_C,), jnp.int32),
            pltpu.VMEM((_C,), jnp.int32),
            pltpu.VMEM((_C, H), jnp.float32),
            pltpu.VMEM((16,), jnp.int32),
            pltpu.VMEM((16,), jnp.int32),
            pltpu.VMEM((16, H), jnp.float32),
            pltpu.SemaphoreType.DMA,
        ],
    )
    def k(p_hbm, q_hbm, src_hbm, dst_hbm, out_hbm,
          idx_s, idx_d, rows, idx_st, idx_dt, rows_t, sem):
        wid = lax.axis_index("s") * NC + lax.axis_index("c")
        base = wid * per

        def body(i, carry):
            off = base + i * _C
            pltpu.sync_copy(src_hbm.at[pl.ds(off, _C)], idx_s)
            pltpu.sync_copy(dst_hbm.at[pl.ds(off, _C)], idx_d)
            pltpu.async_copy(p_hbm.at[idx_s], rows, sem).wait()
            pltpu.async_copy(q_hbm.at[idx_d], rows, sem, add=True).wait()
            pltpu.sync_copy(rows, out_hbm.at[pl.ds(off, _C)])
            return carry

        lax.fori_loop(0, nch, body, 0)
        if tail:
            off = base + nch * _C
            pltpu.sync_copy(src_hbm.at[pl.ds(off, tail)], idx_st)
            pltpu.sync_copy(dst_hbm.at[pl.ds(off, tail)], idx_dt)
            pltpu.async_copy(p_hbm.at[idx_st], rows_t, sem).wait()
            pltpu.async_copy(q_hbm.at[idx_dt], rows_t, sem, add=True).wait()
            pltpu.sync_copy(rows_t, out_hbm.at[pl.ds(off, tail)])

    return k(p, q, src, dst)


def _sc_scatter(e, dst, zeros_nh):
    E, _ = e.shape
    N = zeros_nh.shape[0]
    per = E // NW
    nch = per // _C
    tail = per - nch * _C
    stripe = (N // NS) // 8 * 8
    srem = N - stripe * NS
    mesh = plsc.VectorSubcoreMesh(core_axis_name="c", subcore_axis_name="s")

    @functools.partial(
        pl.kernel,
        out_type=jax.ShapeDtypeStruct((2 * N, H), jnp.float32),
        mesh=mesh,
        scratch_types=[
            pltpu.VMEM((_C,), jnp.int32),
            pltpu.VMEM((_C, H), jnp.float32),
            pltpu.VMEM((16,), jnp.int32),
            pltpu.VMEM((16, H), jnp.float32),
            pltpu.VMEM_SHARED((N, H), jnp.float32),
        ],
    )
    def k(e_hbm, dst_hbm, z_hbm, out_hbm, idx_d, rows, idx_t, rows_t, acc):
        cid = lax.axis_index("c")
        sid = lax.axis_index("s")
        wid = sid * NC + cid
        base = wid * per
        pltpu.sync_copy(z_hbm.at[pl.ds(sid * stripe, stripe)],
                        acc.at[pl.ds(sid * stripe, stripe)])
        if srem:
            @pl.when(sid == NS - 1)
            def _():
                pltpu.sync_copy(z_hbm.at[pl.ds(NS * stripe, srem)],
                                acc.at[pl.ds(NS * stripe, srem)])
        plsc.subcore_barrier()

        def body(i, carry):
            off = base + i * _C
            pltpu.sync_copy(dst_hbm.at[pl.ds(off, _C)], idx_d)
            pltpu.sync_copy(e_hbm.at[pl.ds(off, _C)], rows)
            pltpu.sync_copy(rows, acc.at[idx_d], add=True)
            return carry

        lax.fori_loop(0, nch, body, 0)
        if tail:
            off = base + nch * _C
            pltpu.sync_copy(dst_hbm.at[pl.ds(off, tail)], idx_t)
            pltpu.sync_copy(e_hbm.at[pl.ds(off, tail)], rows_t)
            pltpu.sync_copy(rows_t, acc.at[idx_t], add=True)
        plsc.subcore_barrier()
        pltpu.sync_copy(acc.at[pl.ds(sid * stripe, stripe)],
                        out_hbm.at[pl.ds(cid * N + sid * stripe, stripe)])
        if srem:
            @pl.when(sid == NS - 1)
            def _():
                pltpu.sync_copy(acc.at[pl.ds(NS * stripe, srem)],
                                out_hbm.at[pl.ds(cid * N + NS * stripe, srem)])

    return k(e, dst, zeros_nh)


def kernel(x, node_type, edge_index, edge_attr, params):
    N = x.shape[0]
    E = edge_attr.shape[0]

    kn = jax.random.key(1)
    k1, k2 = jax.random.split(kn)
    temp_noise = 0.5 + 0.1 * jax.random.normal(k1, (N, 1), dtype=x.dtype)
    volt_noise = 0.1 + 0.05 * jax.random.normal(k2, (N, 1), dtype=x.dtype)
    x = x.at[:, 0:1].add(temp_noise)
    x = x.at[:, 1:2].add(volt_noise)

    oh = jax.nn.one_hot(jnp.reshape(node_type, (-1,)), 3, dtype=x.dtype)
    xin = jnp.concatenate([x, oh, jnp.zeros((N, 1), x.dtype)], axis=-1)

    (We1, be1), (We2, be2) = params['node_enc']
    We1p = jnp.concatenate([We1, jnp.zeros((1, H), We1.dtype)], axis=0)
    g0, bb0 = params['node_enc_ln']
    h = _encoder(xin, We1p, be1, We2, be2, g0, bb0, R=2000)

    (Wa1, ba1), (Wa2, ba2) = params['edge_enc']
    ea = jnp.concatenate([edge_attr, jnp.zeros((E, 4), edge_attr.dtype)],
                         axis=-1)
    Wa1p = jnp.concatenate([Wa1, jnp.zeros((4, H), Wa1.dtype)], axis=0)
    ge, bbe = params['edge_enc_ln']
    e = _encoder(ea, Wa1p, ba1, Wa2, ba2, ge, bbe, R=1000)

    src = edge_index[0]
    dst = edge_index[1]
    zeros_nh = jnp.zeros((N, H), jnp.float32)

    for blk in params['blocks']:
        (W1, b1), (W2, b2) = blk['edge_mlp']
        W1e, W1s, W1d = W1[:H], W1[H:2 * H], W1[2 * H:]
        p, q = _proj(h, W1s, W1d, R=2000)
        gsum = _sc_gather(p, q, src, dst)
        eg, ebb = blk['edge_ln']
        e = _edge_update(e, gsum, W1e, b1, W2, b2, eg, ebb, R=1000)
        agg2 = _sc_scatter(e, dst, zeros_nh)
        (V1, c1), (V2, c2) = blk['node_mlp']
        Vh, Va = V1[:H], V1[H:]
        ng, nbb = blk['node_ln']
        h = _node_update(h, agg2, Vh, Va, c1, V2, c2, ng, nbb, R=2000)

    (D1, d1), (D2, d2) = params['decoder']
    D2p = jnp.concatenate([D2, jnp.zeros((H, H - 2), D2.dtype)], axis=1)
    d2p = jnp.concatenate([d2, jnp.zeros((H - 2,), d2.dtype)])
    xres = jnp.pad(x[:, :2], ((0, 0), (0, H - 2)))
    outf = _decoder(h, D1, d1, D2p, d2p, xres, R=2000)
    return outf[:, :2]
